# baseline (speedup 1.0000x reference)
import functools

import jax
import jax.numpy as jnp
from jax import lax
from jax.experimental import pallas as pl
from jax.experimental.pallas import tpu as pltpu
from jax.experimental.pallas import tpu_sc as plsc

K = 8192
HIDDEN = 256
QHID = 64
B, T = 8, 576
N = B * T
TN = 576
NB = N // TN

NW = 32
BPW = N // NW


def _dot_t(a, b):
    return lax.dot_general(a, b, (((1,), (1,)), ((), ())),
                           preferred_element_type=jnp.float32)


def _proj_body(x_ref, win_ref, bin_ref, cb_ref, wc_ref, bc_ref,
               li_ref, lcq_ref):
    li_ref[...] = _dot_t(x_ref[...], win_ref[...]) + bin_ref[...]
    lcq_ref[...] = _dot_t(cb_ref[...], wc_ref[...]) + bc_ref[...]


def _proj(xf, codebook, W_in, b_in, W_code, b_code):
    return pl.pallas_call(
        _proj_body,
        out_shape=[
            jax.ShapeDtypeStruct((N, QHID), jnp.float32),
            jax.ShapeDtypeStruct((K, QHID), jnp.float32),
        ],
    )(xf, W_in, b_in.reshape(1, QHID), codebook, W_code, b_code.reshape(1, QHID))


def _argmin_body(lin_ref, a2_ref, lcn_ref, b2_ref,
                 idx_ref, lp_ref, perp_ref, cnt_s):
    i = pl.program_id(0)

    @pl.when(i == 0)
    def _():
        cnt_s[...] = jnp.zeros((1, K), jnp.float32)

    m = _dot_t(lin_ref[...], lcn_ref[...])
    d = (a2_ref[...] - 2.0 * m) + b2_ref[...]
    mn = jnp.min(d, axis=1, keepdims=True)
    eqf = jnp.where(d == mn, 1.0, 0.0)
    rowcnt = _dot_t(eqf, jnp.ones((1, K), jnp.float32))
    any_tie = jnp.max(rowcnt) > 1.5

    iota_k = lax.broadcasted_iota(jnp.int32, (1, K), 1)
    hi_f = (iota_k >> 6).astype(jnp.float32)
    lo_f = (iota_k & 63).astype(jnp.float32)

    def _fast():
        idxf = 64.0 * _dot_t(eqf, hi_f) + _dot_t(eqf, lo_f)
        idxs = idxf[:, 0].astype(jnp.int32)
        cnt = jnp.dot(jnp.ones((1, TN), jnp.float32), eqf,
                      preferred_element_type=jnp.float32)
        return idxs, cnt

    def _slow():
        iot = lax.broadcasted_iota(jnp.int32, (TN, K), 1)
        idxs = jnp.min(jnp.where(d == mn, iot, K), axis=1)
        onehot = jnp.where(iot == idxs[:, None], 1.0, 0.0)
        cnt = jnp.dot(jnp.ones((1, TN), jnp.float32), onehot,
                      preferred_element_type=jnp.float32)
        return idxs, cnt

    idxs, cnt_inc = lax.cond(any_tie, _slow, _fast)
    idx_ref[0, 0, :] = idxs
    cnt_s[...] += cnt_inc

    @pl.when(i == NB - 1)
    def _():
        probs = cnt_s[...] / float(N)
        lp = -jnp.sum(probs * jnp.log(probs + 1e-6))
        lp_ref[...] = lp.reshape(1, 1)
        perp_ref[...] = jnp.exp(lp).reshape(1, 1)


def _argmin(lin, a2, lcn, b2row):
    return pl.pallas_call(
        _argmin_body,
        grid=(NB,),
        in_specs=[
            pl.BlockSpec((TN, QHID), lambda i: (i, 0)),
            pl.BlockSpec((TN, 1), lambda i: (i, 0)),
            pl.BlockSpec((K, QHID), lambda i: (0, 0)),
            pl.BlockSpec((1, K), lambda i: (0, 0)),
        ],
        out_specs=[
            pl.BlockSpec((1, 1, TN), lambda i: (i, 0, 0)),
            pl.BlockSpec((1, 1), lambda i: (0, 0)),
            pl.BlockSpec((1, 1), lambda i: (0, 0)),
        ],
        out_shape=[
            jax.ShapeDtypeStruct((NB, 1, TN), jnp.int32),
            jax.ShapeDtypeStruct((1, 1), jnp.float32),
            jax.ShapeDtypeStruct((1, 1), jnp.float32),
        ],
        scratch_shapes=[pltpu.VMEM((1, K), jnp.float32)],
    )(lin, a2, lcn, b2row)


def _sc_body(idx_hbm, cb_hbm, out_hbm, idx_v, rows_v, sem):
    wid = lax.axis_index("s") * 2 + lax.axis_index("c")
    base = wid * BPW
    pltpu.sync_copy(idx_hbm.at[pl.ds(base, BPW)], idx_v)
    pltpu.async_copy(cb_hbm.at[idx_v], rows_v, sem).wait()
    pltpu.sync_copy(rows_v, out_hbm.at[pl.ds(base, BPW)])


def _stage_sc(idx, codebook):
    mesh = plsc.VectorSubcoreMesh(core_axis_name="c", subcore_axis_name="s")
    fn = functools.partial(
        pl.kernel,
        mesh=mesh,
        out_type=jax.ShapeDtypeStruct((N, HIDDEN), jnp.float32),
        scratch_types=[
            pltpu.VMEM((BPW,), jnp.int32),
            pltpu.VMEM((BPW, HIDDEN), jnp.float32),
            pltpu.SemaphoreType.DMA,
        ],
    )(_sc_body)
    return fn(idx, codebook)


def _fin_body(x_ref, q_ref, xl_ref, wc_ref, bc_ref, lp_ref, loss_ref):
    d1 = q_ref[...] - x_ref[...]
    s1 = jnp.sum(d1 * d1)
    lq = _dot_t(q_ref[...], wc_ref[...]) + bc_ref[...]
    d2 = lq - xl_ref[...]
    s2 = jnp.sum(d2 * d2)
    loss = 1.25 * (s1 / float(N * HIDDEN) + s2 / float(N * QHID))
    loss_ref[...] = loss.reshape(1, 1) + 0.1 * lp_ref[...]


def _finisher(xf, quantized, xl, W_code, b_code, lp):
    return pl.pallas_call(
        _fin_body,
        out_shape=jax.ShapeDtypeStruct((1, 1), jnp.float32),
    )(xf, quantized, xl, W_code, b_code.reshape(1, QHID), lp)


def kernel(x, codebook, W_in, b_in, W_code, b_code):
    xf = x.reshape(N, HIDDEN)
    li, lcq = _proj(xf, codebook, W_in, b_in, W_code, b_code)
    lin = li * (jnp.sum(li * li, axis=-1, keepdims=True) + 1e-6) ** -0.5
    lcn = lcq * (jnp.sum(lcq * lcq, axis=1, keepdims=True) + 1e-6) ** -0.5
    a2 = jnp.sum(lin ** 2, axis=1, keepdims=True)
    b2row = jnp.sum(lcn ** 2, axis=1)[None, :]
    idx3, lp, perp = _argmin(lin, a2, lcn, b2row)
    idx = idx3.reshape(N)
    quantized = _stage_sc(idx, codebook)
    loss = _finisher(xf, quantized, li, W_code, b_code, lp)
    return (quantized.reshape(B, T, HIDDEN), loss[0, 0], perp[0, 0],
            idx.reshape(B, T))


# speedup vs baseline: 1.2791x; 1.2791x over previous
import functools

import jax
import jax.numpy as jnp
from jax import lax
from jax.experimental import pallas as pl
from jax.experimental.pallas import tpu as pltpu
from jax.experimental.pallas import tpu_sc as plsc

K = 8192
HIDDEN = 256
QHID = 64
B, T = 8, 576
N = B * T
TN = 1152
NB = N // TN

NW = 32
BPW = N // NW


def _dot_t(a, b):
    return lax.dot_general(a, b, (((1,), (1,)), ((), ())),
                           preferred_element_type=jnp.float32)


def _proj_body(x_ref, win_ref, bin_ref, cb_ref, wc_ref, bc_ref,
               li_ref, lcq_ref):
    li_ref[...] = _dot_t(x_ref[...], win_ref[...]) + bin_ref[...]
    lcq_ref[...] = _dot_t(cb_ref[...], wc_ref[...]) + bc_ref[...]


def _proj(xf, codebook, W_in, b_in, W_code, b_code):
    return pl.pallas_call(
        _proj_body,
        out_shape=[
            jax.ShapeDtypeStruct((N, QHID), jnp.float32),
            jax.ShapeDtypeStruct((K, QHID), jnp.float32),
        ],
    )(xf, W_in, b_in.reshape(1, QHID), codebook, W_code, b_code.reshape(1, QHID))


def _argmin_body(lin_ref, a2_ref, lcn_ref, b2_ref,
                 idx_ref, lp_ref, perp_ref, cnt_s):
    i = pl.program_id(0)

    @pl.when(i == 0)
    def _():
        cnt_s[...] = jnp.zeros((1, K), jnp.float32)

    m = _dot_t(lin_ref[...], lcn_ref[...])
    d = (a2_ref[...] - 2.0 * m) + b2_ref[...]
    mn = jnp.min(d, axis=1, keepdims=True)
    iot = lax.broadcasted_iota(jnp.int32, (TN, K), 1)
    idxs = jnp.min(jnp.where(d == mn, iot, K), axis=1)
    idx_ref[0, 0, :] = idxs
    onehot = jnp.where(iot == idxs[:, None], 1.0, 0.0)
    cnt_s[...] += jnp.dot(jnp.ones((1, TN), jnp.float32), onehot,
                          preferred_element_type=jnp.float32)

    @pl.when(i == NB - 1)
    def _():
        probs = cnt_s[...] / float(N)
        lp = -jnp.sum(probs * jnp.log(probs + 1e-6))
        lp_ref[...] = lp.reshape(1, 1)
        perp_ref[...] = jnp.exp(lp).reshape(1, 1)


def _argmin(lin, a2, lcn, b2row):
    return pl.pallas_call(
        _argmin_body,
        grid=(NB,),
        in_specs=[
            pl.BlockSpec((TN, QHID), lambda i: (i, 0)),
            pl.BlockSpec((TN, 1), lambda i: (i, 0)),
            pl.BlockSpec((K, QHID), lambda i: (0, 0)),
            pl.BlockSpec((1, K), lambda i: (0, 0)),
        ],
        out_specs=[
            pl.BlockSpec((1, 1, TN), lambda i: (i, 0, 0)),
            pl.BlockSpec((1, 1), lambda i: (0, 0)),
            pl.BlockSpec((1, 1), lambda i: (0, 0)),
        ],
        out_shape=[
            jax.ShapeDtypeStruct((NB, 1, TN), jnp.int32),
            jax.ShapeDtypeStruct((1, 1), jnp.float32),
            jax.ShapeDtypeStruct((1, 1), jnp.float32),
        ],
        scratch_shapes=[pltpu.VMEM((1, K), jnp.float32)],
    )(lin, a2, lcn, b2row)


def _sc_body(idx_hbm, cb_hbm, out_hbm, idx_v, rows_v, sem):
    wid = lax.axis_index("s") * 2 + lax.axis_index("c")
    base = wid * BPW
    pltpu.sync_copy(idx_hbm.at[pl.ds(base, BPW)], idx_v)
    pltpu.async_copy(cb_hbm.at[idx_v], rows_v, sem).wait()
    pltpu.sync_copy(rows_v, out_hbm.at[pl.ds(base, BPW)])


def _stage_sc(idx, codebook):
    mesh = plsc.VectorSubcoreMesh(core_axis_name="c", subcore_axis_name="s")
    fn = functools.partial(
        pl.kernel,
        mesh=mesh,
        out_type=jax.ShapeDtypeStruct((N, HIDDEN), jnp.float32),
        scratch_types=[
            pltpu.VMEM((BPW,), jnp.int32),
            pltpu.VMEM((BPW, HIDDEN), jnp.float32),
            pltpu.SemaphoreType.DMA,
        ],
    )(_sc_body)
    return fn(idx, codebook)


def _fin_body(x_ref, q_ref, xl_ref, wc_ref, bc_ref, lp_ref, loss_ref):
    d1 = q_ref[...] - x_ref[...]
    s1 = jnp.sum(d1 * d1)
    lq = _dot_t(q_ref[...], wc_ref[...]) + bc_ref[...]
    d2 = lq - xl_ref[...]
    s2 = jnp.sum(d2 * d2)
    loss = 1.25 * (s1 / float(N * HIDDEN) + s2 / float(N * QHID))
    loss_ref[...] = loss.reshape(1, 1) + 0.1 * lp_ref[...]


def _finisher(xf, quantized, xl, W_code, b_code, lp):
    return pl.pallas_call(
        _fin_body,
        out_shape=jax.ShapeDtypeStruct((1, 1), jnp.float32),
    )(xf, quantized, xl, W_code, b_code.reshape(1, QHID), lp)


def kernel(x, codebook, W_in, b_in, W_code, b_code):
    xf = x.reshape(N, HIDDEN)
    li, lcq = _proj(xf, codebook, W_in, b_in, W_code, b_code)
    lin = li * (jnp.sum(li * li, axis=-1, keepdims=True) + 1e-6) ** -0.5
    lcn = lcq * (jnp.sum(lcq * lcq, axis=1, keepdims=True) + 1e-6) ** -0.5
    a2 = jnp.sum(lin ** 2, axis=1, keepdims=True)
    b2row = jnp.sum(lcn ** 2, axis=1)[None, :]
    idx3, lp, perp = _argmin(lin, a2, lcn, b2row)
    idx = idx3.reshape(N)
    quantized = _stage_sc(idx, codebook)
    loss = _finisher(xf, quantized, li, W_code, b_code, lp)
    return (quantized.reshape(B, T, HIDDEN), loss[0, 0], perp[0, 0],
            idx.reshape(B, T))
